# baseline (speedup 1.0000x reference)
import functools

import jax
import jax.numpy as jnp
from jax import lax
from jax.experimental import pallas as pl
from jax.experimental.pallas import tpu as pltpu
from jax.experimental.pallas import tpu_sc as plsc

_NC, _NS, _L = 2, 16, 16
_NW = _NC * _NS

_R, _C = 16384, 100
_RPW = _R // _NW
_RB = 64
_NB = _RPW // _RB

_mesh = plsc.VectorSubcoreMesh(core_axis_name="c", subcore_axis_name="s")


@functools.partial(
    pl.kernel,
    out_type=jax.ShapeDtypeStruct((_R, _C), jnp.int32),
    mesh=_mesh,
    scratch_types=[
        pltpu.VMEM((_RPW, _C), jnp.int32),
        pltpu.VMEM((_RPW, _C), jnp.float32),
        pltpu.SemaphoreType.DMA,
        pltpu.SemaphoreType.DMA,
    ],
    compiler_params=pltpu.CompilerParams(needs_layout_passes=False),
)
def _gather_mask(idx_hbm, table_hbm, out_hbm, idx_v, vals_v, sem, osem):
    wid = lax.axis_index("s") * _NC + lax.axis_index("c")
    r0 = wid * jnp.int32(_RPW)

    pltpu.sync_copy(idx_hbm.at[pl.ds(r0, _RPW), :], idx_v)

    @pl.loop(jnp.int32(0), jnp.int32(_RPW))
    def _(r):
        pltpu.async_copy(table_hbm.at[idx_v.at[r]], vals_v.at[r], sem)

    @pl.loop(jnp.int32(0), jnp.int32(_NB))
    def _(b):
        rb = b * jnp.int32(_RB)

        @pl.loop(jnp.int32(0), jnp.int32(_RB))
        def _(i):
            r = rb + i
            pltpu.make_async_copy(table_hbm.at[idx_v.at[r]], vals_v.at[r], sem).wait()

        @pl.loop(jnp.int32(0), jnp.int32(_RB))
        def _(i):
            r = rb + i
            for c in (0, 16, 32, 48, 64, 80, _C - _L):
                sl = pl.ds(jnp.int32(c), _L)
                idx_v[r, sl] = jnp.int32(1) - vals_v[r, sl].astype(jnp.int32)

        pltpu.async_copy(
            idx_v.at[pl.ds(rb, _RB), :], out_hbm.at[pl.ds(r0 + rb, _RB), :], osem
        )

    @pl.loop(jnp.int32(0), jnp.int32(_NB))
    def _(b):
        rb = b * jnp.int32(_RB)
        pltpu.make_async_copy(
            idx_v.at[pl.ds(rb, _RB), :], out_hbm.at[pl.ds(r0 + rb, _RB), :], osem
        ).wait()


def kernel(donors_idx, mask_fit_X_col):
    idx32 = donors_idx.astype(jnp.int32)
    out = _gather_mask(idx32, mask_fit_X_col)
    return out.astype(donors_idx.dtype)


# speedup vs baseline: 1.0438x; 1.0438x over previous
import functools

import jax
import jax.numpy as jnp
from jax import lax
from jax.experimental import pallas as pl
from jax.experimental.pallas import tpu as pltpu
from jax.experimental.pallas import tpu_sc as plsc

_NC, _NS, _L = 2, 16, 16
_NW = _NC * _NS

_R, _C = 16384, 100
_RPW = _R // _NW

_mesh = plsc.VectorSubcoreMesh(core_axis_name="c", subcore_axis_name="s")


@functools.partial(
    pl.kernel,
    out_type=jax.ShapeDtypeStruct((_R, _C), jnp.int32),
    mesh=_mesh,
    scratch_types=[
        pltpu.VMEM((_RPW, _C), jnp.int32),
        pltpu.VMEM((_RPW, _C), jnp.float32),
        pltpu.SemaphoreType.DMA,
    ],
    compiler_params=pltpu.CompilerParams(needs_layout_passes=False),
)
def _gather_mask(idx_hbm, table_hbm, out_hbm, idx_v, vals_v, sem):
    wid = lax.axis_index("s") * _NC + lax.axis_index("c")
    r0 = wid * jnp.int32(_RPW)

    pltpu.sync_copy(idx_hbm.at[pl.ds(r0, _RPW), :], idx_v)

    @pl.loop(jnp.int32(0), jnp.int32(_RPW), step=jnp.int32(4))
    def _(r):
        for j in range(4):
            rj = r + jnp.int32(j)
            pltpu.async_copy(table_hbm.at[idx_v.at[rj]], vals_v.at[rj], sem)

    @pl.loop(jnp.int32(0), jnp.int32(_RPW), step=jnp.int32(4))
    def _(r):
        for j in range(4):
            rj = r + jnp.int32(j)
            pltpu.make_async_copy(table_hbm.at[idx_v.at[rj]], vals_v.at[rj], sem).wait()

    @pl.loop(jnp.int32(0), jnp.int32(_RPW))
    def _(r):
        for c in (0, 16, 32, 48, 64, 80, _C - _L):
            sl = pl.ds(jnp.int32(c), _L)
            idx_v[r, sl] = jnp.int32(1) - vals_v[r, sl].astype(jnp.int32)

    pltpu.sync_copy(idx_v, out_hbm.at[pl.ds(r0, _RPW), :])


def kernel(donors_idx, mask_fit_X_col):
    idx32 = donors_idx.astype(jnp.int32)
    out = _gather_mask(idx32, mask_fit_X_col)
    return out.astype(donors_idx.dtype)


# speedup vs baseline: 1.0454x; 1.0015x over previous
import functools

import jax
import jax.numpy as jnp
from jax import lax
from jax.experimental import pallas as pl
from jax.experimental.pallas import tpu as pltpu
from jax.experimental.pallas import tpu_sc as plsc

_NC, _NS, _L = 2, 16, 16
_NW = _NC * _NS

_R, _C = 16384, 100
_RPW = _R // _NW

_mesh = plsc.VectorSubcoreMesh(core_axis_name="c", subcore_axis_name="s")


@functools.partial(
    pl.kernel,
    out_type=jax.ShapeDtypeStruct((_R, _C), jnp.int32),
    mesh=_mesh,
    scratch_types=[
        pltpu.VMEM((_RPW, _C), jnp.int32),
        pltpu.VMEM((_RPW, _C), jnp.float32),
        pltpu.SemaphoreType.DMA,
    ],
    compiler_params=pltpu.CompilerParams(needs_layout_passes=False),
)
def _gather_mask(idx_hbm, table_hbm, out_hbm, idx_v, vals_v, sem):
    wid = lax.axis_index("s") * _NC + lax.axis_index("c")
    r0 = wid * jnp.int32(_RPW)

    with jax.named_scope("copyin"):
        pltpu.sync_copy(idx_hbm.at[pl.ds(r0, _RPW), :], idx_v)

    with jax.named_scope("fire"):
        @pl.loop(jnp.int32(0), jnp.int32(_RPW), step=jnp.int32(4))
        def _(r):
            for j in range(4):
                rj = r + jnp.int32(j)
                pltpu.async_copy(table_hbm.at[idx_v.at[rj]], vals_v.at[rj], sem)

    with jax.named_scope("drain"):
        @pl.loop(jnp.int32(0), jnp.int32(_RPW), step=jnp.int32(4))
        def _(r):
            for j in range(4):
                rj = r + jnp.int32(j)
                pltpu.make_async_copy(table_hbm.at[idx_v.at[rj]], vals_v.at[rj], sem).wait()

    with jax.named_scope("subtract"):
        @pl.loop(jnp.int32(0), jnp.int32(_RPW))
        def _(r):
            for c in (0, 16, 32, 48, 64, 80, _C - _L):
                sl = pl.ds(jnp.int32(c), _L)
                idx_v[r, sl] = jnp.int32(1) - vals_v[r, sl].astype(jnp.int32)

    with jax.named_scope("copyout"):
        pltpu.sync_copy(idx_v, out_hbm.at[pl.ds(r0, _RPW), :])


def kernel(donors_idx, mask_fit_X_col):
    idx32 = donors_idx.astype(jnp.int32)
    out = _gather_mask(idx32, mask_fit_X_col)
    return out.astype(donors_idx.dtype)
